# baseline (speedup 1.0000x reference)
import functools

import jax
import jax.numpy as jnp
from jax import lax
from jax.experimental import pallas as pl
from jax.experimental.pallas import tpu as pltpu
from jax.experimental.pallas import tpu_sc as plsc

DIM = 2048
N_HEADS = 32
HEAD_DIM = 128
ROPE_HD = 64
Q_LORA = 1536
SEQ = 2048
EPS = 1e-6
SCALE = HEAD_DIM ** -0.5 * N_HEADS ** -0.5


def _rope(x, cos, sin):
    xl = jnp.concatenate([x[..., 1:], x[..., :1]], axis=-1)
    xr = jnp.concatenate([x[..., -1:], x[..., :-1]], axis=-1)
    odd = lax.broadcasted_iota(jnp.int32, x.shape, x.ndim - 1) % 2 == 1
    swapped = jnp.where(odd, xr, xl)
    s_signed = jnp.where(odd, sin, -sin)
    return x * cos + swapped * s_signed


def _k_kernel(x_ref, wk_ref, wp_ref, cos_ref, sin_ref, knw_ref, knb_ref,
              k_ref, w_ref):
    x = x_ref[...]
    kp = jnp.dot(x, wk_ref[...], preferred_element_type=jnp.float32)
    mu = jnp.mean(kp, axis=-1, keepdims=True)
    var = jnp.mean((kp - mu) ** 2, axis=-1, keepdims=True)
    k = (kp - mu) / jnp.sqrt(var + EPS) * knw_ref[...] + knb_ref[...]
    k_pe = _rope(k[:, :ROPE_HD], cos_ref[...], sin_ref[...])
    k_ref[...] = jnp.concatenate([k_pe, k[:, ROPE_HD:]], axis=-1)
    w_ref[...] = jnp.dot(x, wp_ref[...], preferred_element_type=jnp.float32)


def _score_kernel(base, qr_ref, wqb_ref, k_ref, w_ref, cos_ref, sin_ref,
                  out_ref, flag_ref):
    i = pl.program_id(0)
    blk = qr_ref.shape[0]
    q = jnp.dot(qr_ref[...], wqb_ref[...], preferred_element_type=jnp.float32)
    q = q.reshape(blk, N_HEADS, HEAD_DIM)
    cos = cos_ref[...][:, None, :]
    sin = sin_ref[...][:, None, :]
    q_pe = _rope(q[:, :, :ROPE_HD], cos, sin)
    q = jnp.concatenate([q_pe, q[:, :, ROPE_HD:]], axis=-1)
    w = w_ref[...] * SCALE
    k = k_ref[...]
    acc = jnp.zeros((blk, SEQ), jnp.float32)
    for h in range(N_HEADS):
        l = jax.nn.relu(
            lax.dot_general(q[:, h, :], k, (((1,), (1,)), ((), ())),
                            preferred_element_type=jnp.float32))
        acc = acc + w[:, h:h + 1] * l
    rows = base + i * blk + lax.broadcasted_iota(jnp.int32, (blk, SEQ), 0)
    cols = lax.broadcasted_iota(jnp.int32, (blk, SEQ), 1)
    causal = cols <= rows
    score = acc + jnp.where(causal, 0.0, -1e9).astype(jnp.float32)
    u = lax.bitcast_convert_type(score, jnp.int32)
    out_ref[...] = jnp.where(score < 0, u, ~u ^ jnp.int32(-2147483648))
    bad_pre = jnp.where(score < -1e9, 1.0, 0.0)
    bad_suf = jnp.where(score != -1e9, 1.0, 0.0)
    bad = jnp.where(causal, bad_pre, bad_suf)
    rows1 = base + i * blk + lax.iota(jnp.int32, blk)
    l16 = (rows1 & ~jnp.int32(15)) + 16
    flag_ref[...] = jnp.where((jnp.sum(bad, axis=1) == 0.0) & (l16 < SEQ), 1, 0)


def _scores(x, qr, cos, sin, wq_b, wk, weights_proj, k_norm_w, k_norm_b):
    kb = 512
    k_full, w_full = pl.pallas_call(
        _k_kernel,
        grid=(SEQ // kb,),
        in_specs=[
            pl.BlockSpec((kb, DIM), lambda i: (i, 0)),
            pl.BlockSpec((DIM, HEAD_DIM), lambda i: (0, 0)),
            pl.BlockSpec((DIM, N_HEADS), lambda i: (0, 0)),
            pl.BlockSpec((kb, ROPE_HD), lambda i: (i, 0)),
            pl.BlockSpec((kb, ROPE_HD), lambda i: (i, 0)),
            pl.BlockSpec((HEAD_DIM,), lambda i: (0,)),
            pl.BlockSpec((HEAD_DIM,), lambda i: (0,)),
        ],
        out_specs=[
            pl.BlockSpec((kb, HEAD_DIM), lambda i: (i, 0)),
            pl.BlockSpec((kb, N_HEADS), lambda i: (i, 0)),
        ],
        out_shape=[
            jax.ShapeDtypeStruct((SEQ, HEAD_DIM), jnp.float32),
            jax.ShapeDtypeStruct((SEQ, N_HEADS), jnp.float32),
        ],
    )(x, wk, weights_proj, cos, sin, k_norm_w, k_norm_b)

    return k_full, w_full


_CH = 256


def _score_chunk(qr_c, wq_b, k_full, w_full, cos_c, sin_c, base):
    sb = 256
    return pl.pallas_call(
        functools.partial(_score_kernel, base),
        grid=(_CH // sb,),
        in_specs=[
            pl.BlockSpec((sb, Q_LORA), lambda i: (i, 0)),
            pl.BlockSpec((Q_LORA, N_HEADS * HEAD_DIM), lambda i: (0, 0)),
            pl.BlockSpec((SEQ, HEAD_DIM), lambda i: (0, 0)),
            pl.BlockSpec((sb, N_HEADS), lambda i: (i, 0)),
            pl.BlockSpec((sb, ROPE_HD), lambda i: (i, 0)),
            pl.BlockSpec((sb, ROPE_HD), lambda i: (i, 0)),
        ],
        out_specs=[
            pl.BlockSpec((sb, SEQ), lambda i: (i, 0)),
            pl.BlockSpec((sb,), lambda i: (i,)),
        ],
        out_shape=[
            jax.ShapeDtypeStruct((_CH, SEQ), jnp.int32),
            jax.ShapeDtypeStruct((_CH,), jnp.int32),
        ],
    )(qr_c, wq_b, k_full, w_full, cos_c, sin_c)


_RPW = _CH // 32


def _sc_sort_body(ro, keys_hbm, flags_hbm, out_hbm, ka, kb, va, vb, vout,
                  flags_v, hist, rank, d_buf, c_buf, base_buf):
    wid = lax.axis_index("s") * 2 + lax.axis_index("c")
    lane = lax.iota(jnp.int32, 16)

    pltpu.sync_copy(flags_hbm, flags_v)

    def init(j, _):
        vout[pl.ds(j * 16, 16)] = j * 16 + lane
        return 0
    lax.fori_loop(0, SEQ // 16, init, 0)

    def do_row(r, _):
        row = wid + 32 * r
        pltpu.sync_copy(keys_hbm.at[row], ka)
        l16 = ((ro + row) & ~jnp.int32(15)) + 16
        fvec = flags_v[pl.ds(row & ~jnp.int32(15), 16)]
        flag = jnp.sum(jnp.where(lane == (row & 15), fvec, 0))
        nv = lax.select(flag > 0, lax.div(l16, 16), jnp.int32(SEQ // 16))

        for p in range(4):
            shift = 8 * p
            src_k = ka if p % 2 == 0 else kb
            dst_k = kb if p % 2 == 0 else ka
            src_v = va if p % 2 == 0 else vb
            dst_v = vout if p == 3 else (vb if p % 2 == 0 else va)

            @plsc.parallel_loop(0, 256, 16, unroll=4)
            def zero(i):
                hist[pl.ds(i, 16)] = jnp.zeros((16,), jnp.int32)

            @plsc.parallel_loop(0, nv * 16, 16, unroll=4)
            def digits_step(i):
                k = src_k[pl.ds(i, 16)]
                d = lax.shift_right_logical(k, shift) & 255
                occ, last = plsc.scan_count(d)
                d_buf[pl.ds(i, 16)] = d
                rank[pl.ds(i, 16)] = occ - 1
                c_buf[pl.ds(i, 16)] = jnp.where(last, occ, 0)

            def hist_step(j, _):
                d = d_buf[pl.ds(j * 16, 16)]
                c = c_buf[pl.ds(j * 16, 16)]
                base_buf[pl.ds(j * 16, 16)] = plsc.load_gather(hist, [d])
                plsc.addupdate_scatter(hist, [d], c, mask=c > 0)
                return 0
            lax.fori_loop(0, nv, hist_step, 0)

            def scan_step(j, carry):
                v = hist[pl.ds(j * 16, 16)]
                c = plsc.cumsum(v)
                hist[pl.ds(j * 16, 16)] = c - v + carry
                return carry + jnp.sum(v)
            lax.fori_loop(0, 16, scan_step, jnp.int32(0))

            @plsc.parallel_loop(0, nv * 16, 16, unroll=4)
            def perm_step(i):
                v = i + lane if p == 0 else src_v[pl.ds(i, 16)]
                d = d_buf[pl.ds(i, 16)]
                dest = (plsc.load_gather(hist, [d]) + rank[pl.ds(i, 16)]
                        + base_buf[pl.ds(i, 16)])
                if p < 3:
                    plsc.store_scatter(dst_k, [dest], src_k[pl.ds(i, 16)])
                plsc.store_scatter(dst_v, [dest], v)

        pltpu.sync_copy(vout, out_hbm.at[row])

        @pl.when(flag == 0)
        def _():
            lax.fori_loop(lax.div(l16, 16), SEQ // 16, init, 0)

        return 0

    lax.fori_loop(0, _RPW, do_row, 0)


def _sc_argsort(keys, flags, ro):
    k = pl.kernel(
        functools.partial(_sc_sort_body, ro),
        out_type=jax.ShapeDtypeStruct((_CH, SEQ), jnp.int32),
        mesh=plsc.VectorSubcoreMesh(core_axis_name="c", subcore_axis_name="s"),
        compiler_params=pltpu.CompilerParams(needs_layout_passes=False),
        scratch_types=[
            pltpu.VMEM((SEQ,), jnp.int32),
            pltpu.VMEM((SEQ,), jnp.int32),
            pltpu.VMEM((SEQ,), jnp.int32),
            pltpu.VMEM((SEQ,), jnp.int32),
            pltpu.VMEM((SEQ,), jnp.int32),
            pltpu.VMEM((_CH,), jnp.int32),
            pltpu.VMEM((256,), jnp.int32),
            pltpu.VMEM((SEQ,), jnp.int32),
            pltpu.VMEM((SEQ,), jnp.int32),
            pltpu.VMEM((SEQ,), jnp.int32),
            pltpu.VMEM((SEQ,), jnp.int32),
        ],
    )
    return k(keys, flags)


def kernel(x, qr, cos, sin, mask, wq_b, wk, weights_proj, k_norm_w, k_norm_b):
    del mask
    k_full, w_full = _scores(x[0], qr[0], cos, sin, wq_b, wk, weights_proj,
                             k_norm_w, k_norm_b)
    qr0 = qr[0]
    outs = [None] * (SEQ // _CH)
    for c in reversed(range(SEQ // _CH)):
        lo = c * _CH
        keys_c, flags_c = _score_chunk(
            qr0[lo:lo + _CH], wq_b, k_full, w_full[lo:lo + _CH],
            cos[lo:lo + _CH], sin[lo:lo + _CH], lo)
        outs[c] = _sc_argsort(keys_c, flags_c, lo)
    return jnp.concatenate(outs, axis=0)[None]


# speedup vs baseline: 1.0392x; 1.0392x over previous
import functools

import jax
import jax.numpy as jnp
from jax import lax
from jax.experimental import pallas as pl
from jax.experimental.pallas import tpu as pltpu
from jax.experimental.pallas import tpu_sc as plsc

DIM = 2048
N_HEADS = 32
HEAD_DIM = 128
ROPE_HD = 64
Q_LORA = 1536
SEQ = 2048
EPS = 1e-6
SCALE = HEAD_DIM ** -0.5 * N_HEADS ** -0.5


def _rope(x, cos, sin):
    xl = jnp.concatenate([x[..., 1:], x[..., :1]], axis=-1)
    xr = jnp.concatenate([x[..., -1:], x[..., :-1]], axis=-1)
    odd = lax.broadcasted_iota(jnp.int32, x.shape, x.ndim - 1) % 2 == 1
    swapped = jnp.where(odd, xr, xl)
    s_signed = jnp.where(odd, sin, -sin)
    return x * cos + swapped * s_signed


def _k_kernel(x_ref, wk_ref, wp_ref, cos_ref, sin_ref, knw_ref, knb_ref,
              k_ref, w_ref):
    x = x_ref[...]
    kp = jnp.dot(x, wk_ref[...], preferred_element_type=jnp.float32)
    mu = jnp.mean(kp, axis=-1, keepdims=True)
    var = jnp.mean((kp - mu) ** 2, axis=-1, keepdims=True)
    k = (kp - mu) / jnp.sqrt(var + EPS) * knw_ref[...] + knb_ref[...]
    k_pe = _rope(k[:, :ROPE_HD], cos_ref[...], sin_ref[...])
    k_ref[...] = jnp.concatenate([k_pe, k[:, ROPE_HD:]], axis=-1)
    w_ref[...] = jnp.dot(x, wp_ref[...], preferred_element_type=jnp.float32)


def _score_kernel(base, qr_ref, wqb_ref, k_ref, w_ref, cos_ref, sin_ref,
                  out_ref, flag_ref):
    i = pl.program_id(0)
    blk = qr_ref.shape[0]
    q = jnp.dot(qr_ref[...], wqb_ref[...], preferred_element_type=jnp.float32)
    q = q.reshape(blk, N_HEADS, HEAD_DIM)
    cos = cos_ref[...][:, None, :]
    sin = sin_ref[...][:, None, :]
    q_pe = _rope(q[:, :, :ROPE_HD], cos, sin)
    q = jnp.concatenate([q_pe, q[:, :, ROPE_HD:]], axis=-1)
    w = w_ref[...] * SCALE
    k = k_ref[...]
    acc = jnp.zeros((blk, SEQ), jnp.float32)
    for h in range(N_HEADS):
        l = jax.nn.relu(
            lax.dot_general(q[:, h, :], k, (((1,), (1,)), ((), ())),
                            preferred_element_type=jnp.float32))
        acc = acc + w[:, h:h + 1] * l
    rows = base + i * blk + lax.broadcasted_iota(jnp.int32, (blk, SEQ), 0)
    cols = lax.broadcasted_iota(jnp.int32, (blk, SEQ), 1)
    causal = cols <= rows
    score = acc + jnp.where(causal, 0.0, -1e9).astype(jnp.float32)
    u = lax.bitcast_convert_type(score, jnp.int32)
    out_ref[...] = jnp.where(score < 0, u, ~u ^ jnp.int32(-2147483648))
    bad_pre = jnp.where(score < -1e9, 1.0, 0.0)
    bad_suf = jnp.where(score != -1e9, 1.0, 0.0)
    bad = jnp.where(causal, bad_pre, bad_suf)
    rows1 = base + i * blk + lax.iota(jnp.int32, blk)
    l16 = (rows1 & ~jnp.int32(15)) + 16
    flag_ref[...] = jnp.where((jnp.sum(bad, axis=1) == 0.0) & (l16 < SEQ), 1, 0)


def _scores(x, qr, cos, sin, wq_b, wk, weights_proj, k_norm_w, k_norm_b):
    kb = 512
    k_full, w_full = pl.pallas_call(
        _k_kernel,
        grid=(SEQ // kb,),
        in_specs=[
            pl.BlockSpec((kb, DIM), lambda i: (i, 0)),
            pl.BlockSpec((DIM, HEAD_DIM), lambda i: (0, 0)),
            pl.BlockSpec((DIM, N_HEADS), lambda i: (0, 0)),
            pl.BlockSpec((kb, ROPE_HD), lambda i: (i, 0)),
            pl.BlockSpec((kb, ROPE_HD), lambda i: (i, 0)),
            pl.BlockSpec((HEAD_DIM,), lambda i: (0,)),
            pl.BlockSpec((HEAD_DIM,), lambda i: (0,)),
        ],
        out_specs=[
            pl.BlockSpec((kb, HEAD_DIM), lambda i: (i, 0)),
            pl.BlockSpec((kb, N_HEADS), lambda i: (i, 0)),
        ],
        out_shape=[
            jax.ShapeDtypeStruct((SEQ, HEAD_DIM), jnp.float32),
            jax.ShapeDtypeStruct((SEQ, N_HEADS), jnp.float32),
        ],
    )(x, wk, weights_proj, cos, sin, k_norm_w, k_norm_b)

    return k_full, w_full


_CH = 512


def _score_chunk(qr_c, wq_b, k_full, w_full, cos_c, sin_c, base):
    sb = 256
    return pl.pallas_call(
        functools.partial(_score_kernel, base),
        grid=(_CH // sb,),
        in_specs=[
            pl.BlockSpec((sb, Q_LORA), lambda i: (i, 0)),
            pl.BlockSpec((Q_LORA, N_HEADS * HEAD_DIM), lambda i: (0, 0)),
            pl.BlockSpec((SEQ, HEAD_DIM), lambda i: (0, 0)),
            pl.BlockSpec((sb, N_HEADS), lambda i: (i, 0)),
            pl.BlockSpec((sb, ROPE_HD), lambda i: (i, 0)),
            pl.BlockSpec((sb, ROPE_HD), lambda i: (i, 0)),
        ],
        out_specs=[
            pl.BlockSpec((sb, SEQ), lambda i: (i, 0)),
            pl.BlockSpec((sb,), lambda i: (i,)),
        ],
        out_shape=[
            jax.ShapeDtypeStruct((_CH, SEQ), jnp.int32),
            jax.ShapeDtypeStruct((_CH,), jnp.int32),
        ],
    )(qr_c, wq_b, k_full, w_full, cos_c, sin_c)


_RPW = _CH // 32


def _sc_sort_body(ro, keys_hbm, flags_hbm, out_hbm, ka, kb, va, vb, vout,
                  flags_v, hist, rank, d_buf, c_buf, base_buf):
    wid = lax.axis_index("s") * 2 + lax.axis_index("c")
    lane = lax.iota(jnp.int32, 16)

    pltpu.sync_copy(flags_hbm, flags_v)

    def init(j, _):
        vout[pl.ds(j * 16, 16)] = j * 16 + lane
        return 0
    lax.fori_loop(0, SEQ // 16, init, 0)

    def do_row(r, _):
        row = wid + 32 * r
        pltpu.sync_copy(keys_hbm.at[row], ka)
        l16 = ((ro + row) & ~jnp.int32(15)) + 16
        fvec = flags_v[pl.ds(row & ~jnp.int32(15), 16)]
        flag = jnp.sum(jnp.where(lane == (row & 15), fvec, 0))
        nv = lax.select(flag > 0, lax.div(l16, 16), jnp.int32(SEQ // 16))

        for p in range(4):
            shift = 8 * p
            src_k = ka if p % 2 == 0 else kb
            dst_k = kb if p % 2 == 0 else ka
            src_v = va if p % 2 == 0 else vb
            dst_v = vout if p == 3 else (vb if p % 2 == 0 else va)

            @plsc.parallel_loop(0, 256, 16, unroll=4)
            def zero(i):
                hist[pl.ds(i, 16)] = jnp.zeros((16,), jnp.int32)

            @plsc.parallel_loop(0, nv * 16, 16, unroll=4)
            def digits_step(i):
                k = src_k[pl.ds(i, 16)]
                d = lax.shift_right_logical(k, shift) & 255
                occ, last = plsc.scan_count(d)
                d_buf[pl.ds(i, 16)] = d
                rank[pl.ds(i, 16)] = occ - 1
                c_buf[pl.ds(i, 16)] = jnp.where(last, occ, 0)

            def hist_step(j, _):
                d = d_buf[pl.ds(j * 16, 16)]
                c = c_buf[pl.ds(j * 16, 16)]
                base_buf[pl.ds(j * 16, 16)] = plsc.load_gather(hist, [d])
                plsc.addupdate_scatter(hist, [d], c, mask=c > 0)
                return 0
            lax.fori_loop(0, nv, hist_step, 0)

            def scan_step(j, carry):
                v = hist[pl.ds(j * 16, 16)]
                c = plsc.cumsum(v)
                hist[pl.ds(j * 16, 16)] = c - v + carry
                return carry + jnp.sum(v)
            lax.fori_loop(0, 16, scan_step, jnp.int32(0))

            @plsc.parallel_loop(0, nv * 16, 16, unroll=4)
            def perm_step(i):
                v = i + lane if p == 0 else src_v[pl.ds(i, 16)]
                d = d_buf[pl.ds(i, 16)]
                dest = (plsc.load_gather(hist, [d]) + rank[pl.ds(i, 16)]
                        + base_buf[pl.ds(i, 16)])
                if p < 3:
                    plsc.store_scatter(dst_k, [dest], src_k[pl.ds(i, 16)])
                plsc.store_scatter(dst_v, [dest], v)

        pltpu.sync_copy(vout, out_hbm.at[row])

        @pl.when(flag == 0)
        def _():
            lax.fori_loop(lax.div(l16, 16), SEQ // 16, init, 0)

        return 0

    lax.fori_loop(0, _RPW, do_row, 0)


def _sc_argsort(keys, flags, ro):
    k = pl.kernel(
        functools.partial(_sc_sort_body, ro),
        out_type=jax.ShapeDtypeStruct((_CH, SEQ), jnp.int32),
        mesh=plsc.VectorSubcoreMesh(core_axis_name="c", subcore_axis_name="s"),
        compiler_params=pltpu.CompilerParams(needs_layout_passes=False),
        scratch_types=[
            pltpu.VMEM((SEQ,), jnp.int32),
            pltpu.VMEM((SEQ,), jnp.int32),
            pltpu.VMEM((SEQ,), jnp.int32),
            pltpu.VMEM((SEQ,), jnp.int32),
            pltpu.VMEM((SEQ,), jnp.int32),
            pltpu.VMEM((_CH,), jnp.int32),
            pltpu.VMEM((256,), jnp.int32),
            pltpu.VMEM((SEQ,), jnp.int32),
            pltpu.VMEM((SEQ,), jnp.int32),
            pltpu.VMEM((SEQ,), jnp.int32),
            pltpu.VMEM((SEQ,), jnp.int32),
        ],
    )
    return k(keys, flags)


def kernel(x, qr, cos, sin, mask, wq_b, wk, weights_proj, k_norm_w, k_norm_b):
    del mask
    k_full, w_full = _scores(x[0], qr[0], cos, sin, wq_b, wk, weights_proj,
                             k_norm_w, k_norm_b)
    qr0 = qr[0]
    outs = [None] * (SEQ // _CH)
    for c in reversed(range(SEQ // _CH)):
        lo = c * _CH
        keys_c, flags_c = _score_chunk(
            qr0[lo:lo + _CH], wq_b, k_full, w_full[lo:lo + _CH],
            cos[lo:lo + _CH], sin[lo:lo + _CH], lo)
        outs[c] = _sc_argsort(keys_c, flags_c, lo)
    return jnp.concatenate(outs, axis=0)[None]
